# Initial kernel scaffold; baseline (speedup 1.0000x reference)
import jax
import jax.numpy as jnp
from jax.experimental import pallas as pl


def kernel(position, page, near_expo_seq_cate2, near_expo_seq_cate3, neighbourhood_table, position_table, page_table, W1, b1, g1, be1, W2, b2, g2, be2):
    raise NotImplementedError("write your pallas kernel here")


# speedup vs baseline: 1.1058x; 1.1058x over previous
import functools

import jax
import jax.numpy as jnp
from jax import lax
from jax.experimental import pallas as pl
from jax.experimental.pallas import tpu as pltpu
from jax.experimental.pallas import tpu_sc as plsc

B = 16384
EDIM = 16
NB = 7
NF = 2
NSLOT = NF * NB
D_IN = NSLOT + NF * EDIM + EDIM + EDIM
D_PAD = 128

NC = 2
NS = 16
NW = NC * NS

CHUNK = 128

EMB_PER_W = B * NSLOT // NW
POS_PER_W = B // NW
EMB_CHUNKS = EMB_PER_W // CHUNK
POS_CHUNKS = POS_PER_W // CHUNK


def _sc_gather_body(emb_idx, pos_idx, page_idx,
                    emb_tab, pos_tab, page_tab,
                    emb_out, pos_out, page_out,
                    idx_v, rows_v, sem):
  wid = lax.axis_index("s") * NC + lax.axis_index("c")

  def run(idx_hbm, tab_hbm, out_hbm, base, nchunks):
    def body(i, _):
      off = base + i * CHUNK
      pltpu.sync_copy(idx_hbm.at[pl.ds(off, CHUNK)], idx_v)
      pltpu.async_copy(tab_hbm.at[idx_v], rows_v, sem).wait()
      pltpu.sync_copy(rows_v, out_hbm.at[pl.ds(off, CHUNK)])
      return 0
    lax.fori_loop(0, nchunks, body, 0, unroll=False)

  run(emb_idx, emb_tab, emb_out, wid * EMB_PER_W, EMB_CHUNKS)
  run(pos_idx, pos_tab, pos_out, wid * POS_PER_W, POS_CHUNKS)
  run(page_idx, page_tab, page_out, wid * POS_PER_W, POS_CHUNKS)


@functools.lru_cache(maxsize=None)
def _make_sc_gather():
  return pl.kernel(
      _sc_gather_body,
      out_type=(
          jax.ShapeDtypeStruct((B * NSLOT, EDIM), jnp.float32),
          jax.ShapeDtypeStruct((B, EDIM), jnp.float32),
          jax.ShapeDtypeStruct((B, EDIM), jnp.float32),
      ),
      mesh=plsc.VectorSubcoreMesh(core_axis_name="c", subcore_axis_name="s"),
      compiler_params=pltpu.CompilerParams(use_tc_tiling_on_sc=False),
      scratch_types=[
          pltpu.VMEM((CHUNK,), jnp.int32),
          pltpu.VMEM((CHUNK, EDIM), jnp.float32),
          pltpu.SemaphoreType.DMA,
      ],
  )


def _tc_body(emb_ref, pos_ref, page_ref, w1_ref, b1_ref, g1_ref, be1_ref,
             w2_ref, b2_ref, g2_ref, be2_ref, out_ref):
  x = emb_ref[...]
  q0 = x[:, 3 * EDIM:4 * EDIM]
  q1 = x[:, 10 * EDIM:11 * EDIM]
  qq = jnp.concatenate([q0] * NB + [q1] * NB, axis=1)
  prod = x * qq

  r = lax.broadcasted_iota(jnp.int32, (NSLOT * EDIM, NSLOT), 0)
  c = lax.broadcasted_iota(jnp.int32, (NSLOT * EDIM, NSLOT), 1)
  seg = (r // EDIM == c).astype(jnp.float32)
  scores = jax.lax.dot(prod, seg,
                       preferred_element_type=jnp.float32) * (1.0 / 4.0)

  def softmax7(s):
    m = jnp.max(s, axis=-1, keepdims=True)
    e = jnp.exp(s - m)
    return e / jnp.sum(e, axis=-1, keepdims=True)

  w0 = softmax7(scores[:, :NB])
  w1 = softmax7(scores[:, NB:])
  aw = jnp.concatenate([w0, w1], axis=1)

  ao = []
  for f in range(NF):
    acc = jnp.zeros_like(q0)
    for k in range(NB):
      s = f * NB + k
      acc = acc + aw[:, s:s + 1] * x[:, s * EDIM:(s + 1) * EDIM]
    ao.append(acc)

  blk = x.shape[0]
  pad = jnp.zeros((blk, D_PAD - D_IN), dtype=jnp.float32)
  result = jnp.concatenate(
      [aw, ao[0], ao[1], pos_ref[...], page_ref[...], pad], axis=1)

  h = jax.lax.dot(result, w1_ref[...],
                  preferred_element_type=jnp.float32) + b1_ref[...]
  mu = jnp.mean(h, axis=-1, keepdims=True)
  var = jnp.mean((h - mu) ** 2, axis=-1, keepdims=True)
  h = g1_ref[...] * (h - mu) / jnp.sqrt(var + 1e-3) + be1_ref[...]
  h = jnp.maximum(h, 0.0)

  h2 = jnp.sum(h * w2_ref[...], axis=-1, keepdims=True) + b2_ref[...]
  mu2 = jnp.mean(h2, axis=-1, keepdims=True)
  var2 = jnp.mean((h2 - mu2) ** 2, axis=-1, keepdims=True)
  h2 = g2_ref[...] * (h2 - mu2) / jnp.sqrt(var2 + 1e-3) + be2_ref[...]
  out_ref[...] = jnp.maximum(h2, 0.0)


def kernel(position, page, near_expo_seq_cate2, near_expo_seq_cate3,
           neighbourhood_table, position_table, page_table,
           W1, b1, g1, be1, W2, b2, g2, be2):
  ids = jnp.concatenate(
      [near_expo_seq_cate2, near_expo_seq_cate3], axis=1
  ).reshape(-1).astype(jnp.int32)
  pos_idx = position.astype(jnp.int32)
  page_idx = page.astype(jnp.int32)

  emb_rows, pos_rows, page_rows = _make_sc_gather()(
      ids, pos_idx, page_idx,
      neighbourhood_table, position_table, page_table)

  emb_flat = emb_rows.reshape(B, NSLOT * EDIM)

  w1p = jnp.zeros((D_PAD, 8), jnp.float32).at[:D_IN].set(W1)
  blk = 2048
  grid = B // blk
  out = pl.pallas_call(
      _tc_body,
      grid=(grid,),
      in_specs=[
          pl.BlockSpec((blk, NSLOT * EDIM), lambda i: (i, 0)),
          pl.BlockSpec((blk, EDIM), lambda i: (i, 0)),
          pl.BlockSpec((blk, EDIM), lambda i: (i, 0)),
          pl.BlockSpec((D_PAD, 8), lambda i: (0, 0)),
          pl.BlockSpec((1, 8), lambda i: (0, 0)),
          pl.BlockSpec((1, 8), lambda i: (0, 0)),
          pl.BlockSpec((1, 8), lambda i: (0, 0)),
          pl.BlockSpec((1, 8), lambda i: (0, 0)),
          pl.BlockSpec((1, 1), lambda i: (0, 0)),
          pl.BlockSpec((1, 1), lambda i: (0, 0)),
          pl.BlockSpec((1, 1), lambda i: (0, 0)),
      ],
      out_specs=pl.BlockSpec((blk, 1), lambda i: (i, 0)),
      out_shape=jax.ShapeDtypeStruct((B, 1), jnp.float32),
  )(emb_flat, pos_rows, page_rows, w1p,
    b1.reshape(1, 8), g1.reshape(1, 8), be1.reshape(1, 8),
    W2.reshape(1, 8), b2.reshape(1, 1), g2.reshape(1, 1), be2.reshape(1, 1))
  return out


# speedup vs baseline: 1.2068x; 1.0914x over previous
import functools

import jax
import jax.numpy as jnp
from jax import lax
from jax.experimental import pallas as pl
from jax.experimental.pallas import tpu as pltpu
from jax.experimental.pallas import tpu_sc as plsc

B = 16384
EDIM = 16
NB = 7
NF = 2
NSLOT = NF * NB
D_IN = NSLOT + NF * EDIM + EDIM + EDIM
D_PAD = 128

NC = 2
NS = 16
NW = NC * NS

CHUNK = 128

EMB_PER_W = B * NSLOT // NW
POS_PER_W = B // NW
EMB_CHUNKS = EMB_PER_W // CHUNK
POS_CHUNKS = POS_PER_W // CHUNK


GROUP = 1024


def _sc_gather_body(emb_idx, pos_idx, page_idx,
                    emb_tab, pos_tab, page_tab,
                    emb_out, pos_out, page_out,
                    idx_v, pidx_v, buf0, buf1, g0, g1, s0, s1):
  wid = lax.axis_index("s") * NC + lax.axis_index("c")
  base = wid * EMB_PER_W
  pltpu.sync_copy(emb_idx.at[pl.ds(base, EMB_PER_W)], idx_v)

  bufs = (buf0, buf1)
  gsems = (g0, g1)
  ssems = (s0, s1)
  ngroups = EMB_PER_W // GROUP
  per_group = GROUP // CHUNK
  store_handles = [None, None]
  for g in range(ngroups):
    p = g % 2
    if store_handles[p] is not None:
      store_handles[p].wait()
    handles = []
    for j in range(per_group):
      off = g * GROUP + j * CHUNK
      handles.append(pltpu.async_copy(
          emb_tab.at[idx_v.at[pl.ds(off, CHUNK)]],
          bufs[p].at[pl.ds(j * CHUNK, CHUNK)], gsems[p]))
    for h in handles:
      h.wait()
    store_handles[p] = pltpu.async_copy(
        bufs[p], emb_out.at[pl.ds(base + g * GROUP, GROUP)], ssems[p])
  for h in store_handles:
    if h is not None:
      h.wait()

  pbase = wid * POS_PER_W
  for src_idx, tab, out, buf, gsem, ssem in (
      (pos_idx, pos_tab, pos_out, buf0, g0, s0),
      (page_idx, page_tab, page_out, buf1, g1, s1),
  ):
    pltpu.sync_copy(src_idx.at[pl.ds(pbase, POS_PER_W)], pidx_v)
    handles = []
    for j in range(POS_CHUNKS):
      handles.append(pltpu.async_copy(
          tab.at[pidx_v.at[pl.ds(j * CHUNK, CHUNK)]],
          buf.at[pl.ds(j * CHUNK, CHUNK)], gsem))
    for h in handles:
      h.wait()
    pltpu.async_copy(
        buf.at[pl.ds(0, POS_PER_W)], out.at[pl.ds(pbase, POS_PER_W)],
        ssem).wait()


@functools.lru_cache(maxsize=None)
def _make_sc_gather():
  return pl.kernel(
      _sc_gather_body,
      out_type=(
          jax.ShapeDtypeStruct((B * NSLOT, EDIM), jnp.float32),
          jax.ShapeDtypeStruct((B, EDIM), jnp.float32),
          jax.ShapeDtypeStruct((B, EDIM), jnp.float32),
      ),
      mesh=plsc.VectorSubcoreMesh(core_axis_name="c", subcore_axis_name="s"),
      compiler_params=pltpu.CompilerParams(use_tc_tiling_on_sc=False),
      scratch_types=[
          pltpu.VMEM((EMB_PER_W,), jnp.int32),
          pltpu.VMEM((POS_PER_W,), jnp.int32),
          pltpu.VMEM((GROUP, EDIM), jnp.float32),
          pltpu.VMEM((GROUP, EDIM), jnp.float32),
          pltpu.SemaphoreType.DMA,
          pltpu.SemaphoreType.DMA,
          pltpu.SemaphoreType.DMA,
          pltpu.SemaphoreType.DMA,
      ],
  )


def _tc_body(emb_ref, pos_ref, page_ref, w1_ref, b1_ref, g1_ref, be1_ref,
             w2_ref, b2_ref, g2_ref, be2_ref, out_ref):
  x = emb_ref[...]
  q0 = x[:, 3 * EDIM:4 * EDIM]
  q1 = x[:, 10 * EDIM:11 * EDIM]
  qq = jnp.concatenate([q0] * NB + [q1] * NB, axis=1)
  prod = x * qq

  r = lax.broadcasted_iota(jnp.int32, (NSLOT * EDIM, NSLOT), 0)
  c = lax.broadcasted_iota(jnp.int32, (NSLOT * EDIM, NSLOT), 1)
  seg = (r // EDIM == c).astype(jnp.float32)
  scores = jax.lax.dot(prod, seg,
                       preferred_element_type=jnp.float32) * (1.0 / 4.0)

  def softmax7(s):
    m = jnp.max(s, axis=-1, keepdims=True)
    e = jnp.exp(s - m)
    return e / jnp.sum(e, axis=-1, keepdims=True)

  w0 = softmax7(scores[:, :NB])
  w1 = softmax7(scores[:, NB:])
  aw = jnp.concatenate([w0, w1], axis=1)

  ao = []
  for f in range(NF):
    acc = jnp.zeros_like(q0)
    for k in range(NB):
      s = f * NB + k
      acc = acc + aw[:, s:s + 1] * x[:, s * EDIM:(s + 1) * EDIM]
    ao.append(acc)

  blk = x.shape[0]
  pad = jnp.zeros((blk, D_PAD - D_IN), dtype=jnp.float32)
  result = jnp.concatenate(
      [aw, ao[0], ao[1], pos_ref[...], page_ref[...], pad], axis=1)

  h = jax.lax.dot(result, w1_ref[...],
                  preferred_element_type=jnp.float32) + b1_ref[...]
  mu = jnp.mean(h, axis=-1, keepdims=True)
  var = jnp.mean((h - mu) ** 2, axis=-1, keepdims=True)
  h = g1_ref[...] * (h - mu) / jnp.sqrt(var + 1e-3) + be1_ref[...]
  h = jnp.maximum(h, 0.0)

  h2 = jnp.sum(h * w2_ref[...], axis=-1, keepdims=True) + b2_ref[...]
  mu2 = jnp.mean(h2, axis=-1, keepdims=True)
  var2 = jnp.mean((h2 - mu2) ** 2, axis=-1, keepdims=True)
  h2 = g2_ref[...] * (h2 - mu2) / jnp.sqrt(var2 + 1e-3) + be2_ref[...]
  out_ref[...] = jnp.maximum(h2, 0.0)


def kernel(position, page, near_expo_seq_cate2, near_expo_seq_cate3,
           neighbourhood_table, position_table, page_table,
           W1, b1, g1, be1, W2, b2, g2, be2):
  ids = jnp.concatenate(
      [near_expo_seq_cate2, near_expo_seq_cate3], axis=1
  ).reshape(-1).astype(jnp.int32)
  pos_idx = position.astype(jnp.int32)
  page_idx = page.astype(jnp.int32)

  emb_rows, pos_rows, page_rows = _make_sc_gather()(
      ids, pos_idx, page_idx,
      neighbourhood_table, position_table, page_table)

  emb_flat = emb_rows.reshape(B, NSLOT * EDIM)

  w1p = jnp.zeros((D_PAD, 8), jnp.float32).at[:D_IN].set(W1)
  blk = 2048
  grid = B // blk
  out = pl.pallas_call(
      _tc_body,
      grid=(grid,),
      in_specs=[
          pl.BlockSpec((blk, NSLOT * EDIM), lambda i: (i, 0)),
          pl.BlockSpec((blk, EDIM), lambda i: (i, 0)),
          pl.BlockSpec((blk, EDIM), lambda i: (i, 0)),
          pl.BlockSpec((D_PAD, 8), lambda i: (0, 0)),
          pl.BlockSpec((1, 8), lambda i: (0, 0)),
          pl.BlockSpec((1, 8), lambda i: (0, 0)),
          pl.BlockSpec((1, 8), lambda i: (0, 0)),
          pl.BlockSpec((1, 8), lambda i: (0, 0)),
          pl.BlockSpec((1, 1), lambda i: (0, 0)),
          pl.BlockSpec((1, 1), lambda i: (0, 0)),
          pl.BlockSpec((1, 1), lambda i: (0, 0)),
      ],
      out_specs=pl.BlockSpec((blk, 1), lambda i: (i, 0)),
      out_shape=jax.ShapeDtypeStruct((B, 1), jnp.float32),
  )(emb_flat, pos_rows, page_rows, w1p,
    b1.reshape(1, 8), g1.reshape(1, 8), be1.reshape(1, 8),
    W2.reshape(1, 8), b2.reshape(1, 1), g2.reshape(1, 1), be2.reshape(1, 1))
  return out


# speedup vs baseline: 1.3415x; 1.1116x over previous
import functools

import jax
import jax.numpy as jnp
from jax import lax
from jax.experimental import pallas as pl
from jax.experimental.pallas import tpu as pltpu
from jax.experimental.pallas import tpu_sc as plsc

B = 16384
EDIM = 16
NB = 7
NF = 2
NSLOT = NF * NB
D_IN = NSLOT + NF * EDIM + EDIM + EDIM
D_PAD = 128

NC = 2
NS = 16
NW = NC * NS

CHUNK = 128

EMB_PER_W = B * NSLOT // NW
POS_PER_W = B // NW
EMB_CHUNKS = EMB_PER_W // CHUNK
POS_CHUNKS = POS_PER_W // CHUNK


GROUP = 1024


def _sc_gather_body(emb_idx, pos_idx, page_idx,
                    emb_tab, pos_tab, page_tab,
                    emb_out, pos_out, page_out,
                    idx_v, pidx_v, buf0, buf1, g0, g1, s0, s1):
  wid = lax.axis_index("s") * NC + lax.axis_index("c")
  base = wid * EMB_PER_W
  pltpu.sync_copy(emb_idx.at[pl.ds(base, EMB_PER_W)], idx_v)

  bufs = (buf0, buf1)
  gsems = (g0, g1)
  ssems = (s0, s1)
  ngroups = EMB_PER_W // GROUP
  per_group = GROUP // CHUNK
  store_handles = [None, None]
  for g in range(ngroups):
    p = g % 2
    if store_handles[p] is not None:
      store_handles[p].wait()
    handles = []
    for j in range(per_group):
      off = g * GROUP + j * CHUNK
      handles.append(pltpu.async_copy(
          emb_tab.at[idx_v.at[pl.ds(off, CHUNK)]],
          bufs[p].at[pl.ds(j * CHUNK, CHUNK)], gsems[p]))
    for h in handles:
      h.wait()
    store_handles[p] = pltpu.async_copy(
        bufs[p], emb_out.at[pl.ds(base + g * GROUP, GROUP)], ssems[p])
  for h in store_handles:
    if h is not None:
      h.wait()

  pbase = wid * POS_PER_W
  for src_idx, tab, out, buf, gsem, ssem in (
      (pos_idx, pos_tab, pos_out, buf0, g0, s0),
      (page_idx, page_tab, page_out, buf1, g1, s1),
  ):
    pltpu.sync_copy(src_idx.at[pl.ds(pbase, POS_PER_W)], pidx_v)
    handles = []
    for j in range(POS_CHUNKS):
      handles.append(pltpu.async_copy(
          tab.at[pidx_v.at[pl.ds(j * CHUNK, CHUNK)]],
          buf.at[pl.ds(j * CHUNK, CHUNK)], gsem))
    for h in handles:
      h.wait()
    pltpu.async_copy(
        buf.at[pl.ds(0, POS_PER_W)], out.at[pl.ds(pbase, POS_PER_W)],
        ssem).wait()


@functools.lru_cache(maxsize=None)
def _make_sc_gather():
  return pl.kernel(
      _sc_gather_body,
      out_type=(
          jax.ShapeDtypeStruct((B * NSLOT, EDIM), jnp.float32),
          jax.ShapeDtypeStruct((B, EDIM), jnp.float32),
          jax.ShapeDtypeStruct((B, EDIM), jnp.float32),
      ),
      mesh=plsc.VectorSubcoreMesh(core_axis_name="c", subcore_axis_name="s"),
      compiler_params=pltpu.CompilerParams(use_tc_tiling_on_sc=False),
      scratch_types=[
          pltpu.VMEM((EMB_PER_W,), jnp.int32),
          pltpu.VMEM((POS_PER_W,), jnp.int32),
          pltpu.VMEM((GROUP, EDIM), jnp.float32),
          pltpu.VMEM((GROUP, EDIM), jnp.float32),
          pltpu.SemaphoreType.DMA,
          pltpu.SemaphoreType.DMA,
          pltpu.SemaphoreType.DMA,
          pltpu.SemaphoreType.DMA,
      ],
  )


@functools.lru_cache(maxsize=None)
def _selection_mats():
  import numpy as np
  d = NSLOT * EDIM
  i = np.arange(d)
  s = i // EDIM
  e = i % EDIM
  f = s // NB
  qsel = np.zeros((d, d), np.float32)
  qsel[(f * NB + 3) * EDIM + e, i] = 1.0
  seg = np.zeros((d, NSLOT), np.float32)
  seg[i, s] = 1.0
  rsel = seg.T.copy()
  asel = np.zeros((d, NF * EDIM), np.float32)
  asel[i, f * EDIM + e] = 1.0
  return (jnp.asarray(qsel), jnp.asarray(seg),
          jnp.asarray(rsel), jnp.asarray(asel))


def _tc_body(emb_ref, pos_ref, page_ref, qsel_ref, seg_ref, rsel_ref,
             w1aw_ref, w1wx_ref, w1pos_ref, w1page_ref, b1_ref, g1_ref,
             be1_ref, w2_ref, b2_ref, g2_ref, be2_ref, out_ref):
  x = emb_ref[...]
  qq = jax.lax.dot(x, qsel_ref[...], preferred_element_type=jnp.float32)
  prod = x * qq
  scores = jax.lax.dot(prod, seg_ref[...],
                       preferred_element_type=jnp.float32) * (1.0 / 4.0)

  def softmax7(s):
    m = jnp.max(s, axis=-1, keepdims=True)
    e = jnp.exp(s - m)
    return e / jnp.sum(e, axis=-1, keepdims=True)

  w0 = softmax7(scores[:, :NB])
  w1 = softmax7(scores[:, NB:2 * NB])
  aw = jnp.concatenate([w0, w1], axis=1)

  w_rep = jax.lax.dot(aw, rsel_ref[...],
                      preferred_element_type=jnp.float32)
  wx = x * w_rep
  h = (jax.lax.dot(aw, w1aw_ref[...], preferred_element_type=jnp.float32)
       + jax.lax.dot(wx, w1wx_ref[...], preferred_element_type=jnp.float32)
       + jax.lax.dot(pos_ref[...], w1pos_ref[...],
                     preferred_element_type=jnp.float32)
       + jax.lax.dot(page_ref[...], w1page_ref[...],
                     preferred_element_type=jnp.float32)
       + b1_ref[...])
  mu = jnp.mean(h, axis=-1, keepdims=True)
  var = jnp.mean((h - mu) ** 2, axis=-1, keepdims=True)
  h = g1_ref[...] * (h - mu) / jnp.sqrt(var + 1e-3) + be1_ref[...]
  h = jnp.maximum(h, 0.0)

  h2 = jnp.sum(h * w2_ref[...], axis=-1, keepdims=True) + b2_ref[...]
  mu2 = jnp.mean(h2, axis=-1, keepdims=True)
  var2 = jnp.mean((h2 - mu2) ** 2, axis=-1, keepdims=True)
  h2 = g2_ref[...] * (h2 - mu2) / jnp.sqrt(var2 + 1e-3) + be2_ref[...]
  out_ref[...] = jnp.maximum(h2, 0.0)


def kernel(position, page, near_expo_seq_cate2, near_expo_seq_cate3,
           neighbourhood_table, position_table, page_table,
           W1, b1, g1, be1, W2, b2, g2, be2):
  ids = jnp.concatenate(
      [near_expo_seq_cate2, near_expo_seq_cate3], axis=1
  ).reshape(-1).astype(jnp.int32)
  pos_idx = position.astype(jnp.int32)
  page_idx = page.astype(jnp.int32)

  emb_rows, pos_rows, page_rows = _make_sc_gather()(
      ids, pos_idx, page_idx,
      neighbourhood_table, position_table, page_table)

  emb_flat = emb_rows.reshape(B, NSLOT * EDIM)

  qsel, seg, rsel, asel = _selection_mats()
  w1wx = asel @ W1[NSLOT:NSLOT + NF * EDIM]
  w1aw = W1[:NSLOT]
  w1pos = W1[NSLOT + NF * EDIM:NSLOT + NF * EDIM + EDIM]
  w1page = W1[NSLOT + NF * EDIM + EDIM:]

  blk = 2048
  grid = B // blk
  full = lambda i: (0, 0)
  row = lambda i: (i, 0)
  out = pl.pallas_call(
      _tc_body,
      grid=(grid,),
      in_specs=[
          pl.BlockSpec((blk, NSLOT * EDIM), row),
          pl.BlockSpec((blk, EDIM), row),
          pl.BlockSpec((blk, EDIM), row),
          pl.BlockSpec(qsel.shape, full),
          pl.BlockSpec(seg.shape, full),
          pl.BlockSpec(rsel.shape, full),
          pl.BlockSpec((NSLOT, 8), full),
          pl.BlockSpec((NSLOT * EDIM, 8), full),
          pl.BlockSpec((EDIM, 8), full),
          pl.BlockSpec((EDIM, 8), full),
          pl.BlockSpec((1, 8), full),
          pl.BlockSpec((1, 8), full),
          pl.BlockSpec((1, 8), full),
          pl.BlockSpec((1, 8), full),
          pl.BlockSpec((1, 1), full),
          pl.BlockSpec((1, 1), full),
          pl.BlockSpec((1, 1), full),
      ],
      out_specs=pl.BlockSpec((blk, 1), row),
      out_shape=jax.ShapeDtypeStruct((B, 1), jnp.float32),
  )(emb_flat, pos_rows, page_rows, qsel, seg, rsel,
    w1aw, w1wx, w1pos, w1page,
    b1.reshape(1, 8), g1.reshape(1, 8), be1.reshape(1, 8),
    W2.reshape(1, 8), b2.reshape(1, 1), g2.reshape(1, 1), be2.reshape(1, 1))
  return out
